# baseline (speedup 1.0000x reference)
import jax
import jax.numpy as jnp
from jax import lax
from jax.experimental import pallas as pl
from jax.experimental.pallas import tpu as pltpu

E = 8
D = 768
H = 128
N = 16384
BLK = 1024
NBLK = N // BLK
HA = E * H


def _body(lab_ref, x_ref, weall_ref, lanee_ref, wdall_ref, bfull_ref,
          y_ref, loss_ref, acc_ref):
    b = pl.program_id(0)
    x = x_ref[...]
    h_all = jnp.dot(x.astype(jnp.bfloat16), weall_ref[...],
                    preferred_element_type=jnp.float32)

    lab = lab_ref[0]
    maskb = (lanee_ref[...] == lab).astype(jnp.bfloat16)
    hm = h_all.astype(jnp.bfloat16) * maskb
    onehot = (lab == lax.broadcasted_iota(jnp.int32, (BLK, H), 1)
              ).astype(jnp.bfloat16)

    y = (jnp.dot(hm, wdall_ref[...], preferred_element_type=jnp.float32)
         + jnp.dot(onehot, bfull_ref[...],
                   preferred_element_type=jnp.float32))
    y_ref[...] = y

    diff = y - x
    ones = jnp.ones((1, BLK), jnp.float32)
    rowsum = jnp.dot(ones, diff * diff,
                     preferred_element_type=jnp.float32)

    @pl.when(b == 0)
    def _():
        acc_ref[...] = jnp.zeros((1, D), jnp.float32)

    acc_ref[...] += rowsum

    @pl.when(b == NBLK - 1)
    def _():
        loss_ref[...] = jnp.reshape(jnp.sum(acc_ref[...]) / (N * D), (1, 1))


def kernel(img, label, We, be, Wd, bd):
    lab3d = label.astype(jnp.int32).reshape(NBLK, BLK, 1)
    we_all = jnp.transpose(We, (1, 0, 2)).reshape(D, HA).astype(jnp.bfloat16)
    wd_all = Wd.reshape(HA, D).astype(jnp.bfloat16)
    lane_e = (jnp.arange(HA, dtype=jnp.int32) // H).reshape(1, HA)
    bfull = jnp.concatenate(
        [jnp.einsum("eh,ehd->ed", be, Wd) + bd,
         jnp.zeros((H - E, D), jnp.float32)], axis=0).astype(jnp.bfloat16)

    grid_spec = pltpu.PrefetchScalarGridSpec(
        num_scalar_prefetch=0,
        grid=(NBLK,),
        in_specs=[
            pl.BlockSpec((1, BLK, 1), lambda b: (b, 0, 0)),
            pl.BlockSpec((BLK, D), lambda b: (b, 0)),
            pl.BlockSpec((D, HA), lambda b: (0, 0)),
            pl.BlockSpec((1, HA), lambda b: (0, 0)),
            pl.BlockSpec((HA, D), lambda b: (0, 0)),
            pl.BlockSpec((H, D), lambda b: (0, 0)),
        ],
        out_specs=[
            pl.BlockSpec((BLK, D), lambda b: (b, 0)),
            pl.BlockSpec((1, 1), lambda b: (0, 0)),
        ],
        scratch_shapes=[pltpu.VMEM((1, D), jnp.float32)],
    )
    out, loss = pl.pallas_call(
        _body,
        grid_spec=grid_spec,
        out_shape=(
            jax.ShapeDtypeStruct((N, D), jnp.float32),
            jax.ShapeDtypeStruct((1, 1), jnp.float32),
        ),
    )(lab3d, img, we_all, lane_e, wd_all, bfull)
    return loss.reshape(()), out


# speedup vs baseline: 1.0447x; 1.0447x over previous
import jax
import jax.numpy as jnp
from jax import lax
from jax.experimental import pallas as pl
from jax.experimental.pallas import tpu as pltpu

E = 8
D = 768
H = 128
N = 16384
BLK = 1024
NBLK = N // BLK
HA = E * H


def _body(lab_ref, x_ref, we_ref, be_ref, wd_ref, bd_ref, lanee_ref,
          y_ref, loss_ref, wea_scr, wda_scr, bf_scr, acc_ref):
    b = pl.program_id(0)

    @pl.when(b == 0)
    def _():
        for e in range(E):
            wea_scr[:, e * H:(e + 1) * H] = we_ref[e].astype(jnp.bfloat16)
            wda_scr[e * H:(e + 1) * H, :] = wd_ref[e].astype(jnp.bfloat16)
            row = (jnp.dot(be_ref[e:e + 1, :], wd_ref[e],
                           preferred_element_type=jnp.float32)
                   + bd_ref[e:e + 1, :])
            bf_scr[e:e + 1, :] = row.astype(jnp.bfloat16)
        acc_ref[...] = jnp.zeros((1, D), jnp.float32)

    x = x_ref[...]
    h_all = jnp.dot(x.astype(jnp.bfloat16), wea_scr[...],
                    preferred_element_type=jnp.float32)

    lab = lab_ref[0]
    maskb = (lanee_ref[...] == lab).astype(jnp.bfloat16)
    hm = h_all.astype(jnp.bfloat16) * maskb
    onehot = (lab == lax.broadcasted_iota(jnp.int32, (BLK, E), 1)
              ).astype(jnp.bfloat16)

    y = (jnp.dot(hm, wda_scr[...], preferred_element_type=jnp.float32)
         + jnp.dot(onehot, bf_scr[...],
                   preferred_element_type=jnp.float32))
    y_ref[...] = y

    diff = y - x
    ones = jnp.ones((1, BLK), jnp.float32)
    rowsum = jnp.dot(ones, diff * diff,
                     preferred_element_type=jnp.float32)
    acc_ref[...] += rowsum

    @pl.when(b == NBLK - 1)
    def _():
        loss_ref[...] = jnp.reshape(jnp.sum(acc_ref[...]) / (N * D), (1, 1))


def kernel(img, label, We, be, Wd, bd):
    lab3d = label.astype(jnp.int32).reshape(NBLK, BLK, 1)
    lane_e = (jnp.arange(HA, dtype=jnp.int32) // H).reshape(1, HA)

    grid_spec = pltpu.PrefetchScalarGridSpec(
        num_scalar_prefetch=0,
        grid=(NBLK,),
        in_specs=[
            pl.BlockSpec((1, BLK, 1), lambda b: (b, 0, 0)),
            pl.BlockSpec((BLK, D), lambda b: (b, 0)),
            pl.BlockSpec((E, D, H), lambda b: (0, 0, 0)),
            pl.BlockSpec((E, H), lambda b: (0, 0)),
            pl.BlockSpec((E, H, D), lambda b: (0, 0, 0)),
            pl.BlockSpec((E, D), lambda b: (0, 0)),
            pl.BlockSpec((1, HA), lambda b: (0, 0)),
        ],
        out_specs=[
            pl.BlockSpec((BLK, D), lambda b: (b, 0)),
            pl.BlockSpec((1, 1), lambda b: (0, 0)),
        ],
        scratch_shapes=[
            pltpu.VMEM((D, HA), jnp.bfloat16),
            pltpu.VMEM((HA, D), jnp.bfloat16),
            pltpu.VMEM((E, D), jnp.bfloat16),
            pltpu.VMEM((1, D), jnp.float32),
        ],
    )
    out, loss = pl.pallas_call(
        _body,
        grid_spec=grid_spec,
        out_shape=(
            jax.ShapeDtypeStruct((N, D), jnp.float32),
            jax.ShapeDtypeStruct((1, 1), jnp.float32),
        ),
    )(lab3d, img, We, be, Wd, bd, lane_e)
    return loss.reshape(()), out
